# baseline (speedup 1.0000x reference)
import functools
import jax
import jax.numpy as jnp
from jax import lax
from jax.experimental import pallas as pl
from jax.experimental.pallas import tpu as pltpu
from jax.experimental.pallas import tpu_sc as plsc

USER_NUM = 20000
ITEM_NUM = 30000
N = USER_NUM + ITEM_NUM
E = 800000
D = 64
DH = 32
B = 4096
N_LAYERS = 3
LMBD = 1e-4

NC = 2
NS = 16
SB = 128
SB_PER_CHUNK = 16
CHUNK = SB * SB_PER_CHUNK
N_CHUNKS = 25
EC = CHUNK * N_CHUNKS
E_PAD = EC * NS

N_PAD = 50048
ROWS_PER_TILE = N_PAD // NS
ZERO_ROWS = 136

_GDN = None


def _lane_broadcast(v, e):
    idx = jnp.full((16, 1), e, dtype=jnp.int32)
    dnums = lax.GatherDimensionNumbers(
        offset_dims=(), collapsed_slice_dims=(0,), start_index_map=(0,))
    return lax.gather(v, idx, dnums, (1,),
                      mode=lax.GatherScatterMode.PROMISE_IN_BOUNDS)


def _scale_rows(rows_ref, msg_ref, val_ref, vbase):
    for g in range(SB // 16):
        v = val_ref[pl.ds(vbase + g * 16, 16)]
        for e in range(16):
            b = _lane_broadcast(v, e)
            r = g * 16 + e
            for h in range(DH // 16):
                sl = pl.ds(h * 16, 16)
                msg_ref[r, sl] = rows_ref[r, sl] * b


def _propagate_layer_body(table_h, src0_h, src1_h, dst2d_h, val_h, zeros_h,
                          out_h, src_v, dst_v, val_v, rows_v, msg_v, sem,
                          acc_sh):
    c = lax.axis_index("c")
    s = lax.axis_index("s")
    row0 = s * ROWS_PER_TILE

    pltpu.sync_copy(zeros_h.at[pl.ds(row0, ROWS_PER_TILE)],
                    acc_sh.at[pl.ds(row0, ROWS_PER_TILE)])
    plsc.subcore_barrier()

    ebase = s * EC
    msg_a, msg_b = msg_v
    gsems = sem[:4]
    ssem_a, ssem_b = sem[4], sem[5]
    msgs = (msg_a, msg_b)
    ssems = (ssem_a, ssem_b)

    def _gather(j, rowsb, gsem):
        pltpu.async_copy(table_h.at[src_v.at[pl.ds(j * SB, SB)]], rowsb, gsem)

    def _wait_gather(rowsb, gsem):
        pltpu.make_async_copy(table_h.at[src_v.at[pl.ds(0, SB)]], rowsb,
                              gsem).wait()

    def _scatter(j, msgb, ssem):
        pltpu.async_copy(msgb, acc_sh.at[dst_v.at[j]], ssem, add=True)

    def _wait_scatter(msgb, ssem):
        pltpu.make_async_copy(msgb, acc_sh.at[dst_v.at[0]], ssem).wait()

    @pl.loop(0, N_CHUNKS)
    def _chunk(i):
        @pl.when(i > 0)
        def _():
            _wait_scatter(msg_a, ssem_a)
            _wait_scatter(msg_b, ssem_b)

        base = pl.multiple_of(ebase + i * CHUNK, CHUNK)
        row_base = pl.multiple_of(base // SB, SB_PER_CHUNK)

        @pl.when(c == 0)
        def _():
            pltpu.sync_copy(src0_h.at[pl.ds(base, CHUNK)], src_v)

        @pl.when(c == 1)
        def _():
            pltpu.sync_copy(src1_h.at[pl.ds(base, CHUNK)], src_v)

        pltpu.sync_copy(dst2d_h.at[pl.ds(row_base, SB_PER_CHUNK)], dst_v)
        pltpu.sync_copy(val_h.at[pl.ds(base, CHUNK)], val_v)

        for u in range(3):
            _gather(u, rows_v[u], gsems[u])

        @pl.loop(0, SB_PER_CHUNK // 4)
        def _quad(k):
            j = k * 4
            for u in range(4):
                jj = j + u
                _wait_gather(rows_v[u], gsems[u])
                if u < 2:
                    @pl.when(k > 0)
                    def _():
                        _wait_scatter(msgs[u % 2], ssems[u % 2])
                else:
                    _wait_scatter(msgs[u % 2], ssems[u % 2])
                _scale_rows(rows_v[u], msgs[u % 2], val_v, jj * SB)
                _scatter(jj, msgs[u % 2], ssems[u % 2])
                if u == 0:
                    _gather(jj + 3, rows_v[3], gsems[3])
                else:
                    @pl.when(k < SB_PER_CHUNK // 4 - 1)
                    def _():
                        _gather(jj + 3, rows_v[(u + 3) % 4],
                                gsems[(u + 3) % 4])

    _wait_scatter(msg_a, ssem_a)
    _wait_scatter(msg_b, ssem_b)
    plsc.subcore_barrier()

    pltpu.sync_copy(acc_sh.at[pl.ds(row0, ROWS_PER_TILE)],
                    out_h.at[pl.ds(c * N_PAD + row0, ROWS_PER_TILE)])


_BPT = B // NS
_BPW = B // (NC * NS)


def _gather_stage_body(t0_h, t1_h, t2_h, t3_h, uidx_h, iidx_h, nidx_h,
                       uid_h, iid_h, nid_h, uemb_h, iemb_h,
                       ue_h, pe_h, ne_h, uego_h, pego_h, nego_h,
                       idx_v, g_v, idx2_v, ego_v, sem):
    c = lax.axis_index("c")
    s = lax.axis_index("s")

    for set_idx, ids_h, out_h in ((0, uidx_h, ue_h), (1, iidx_h, pe_h),
                                  (2, nidx_h, ne_h)):
        pltpu.sync_copy(ids_h.at[c, pl.ds(s * _BPT, _BPT)], idx_v)

        @pl.loop(0, _BPT // SB)
        def _blk(j):
            isl = idx_v.at[pl.ds(j * SB, SB)]
            pltpu.async_copy(t0_h.at[isl], g_v, sem).wait()
            pltpu.async_copy(t1_h.at[isl], g_v, sem, add=True).wait()
            pltpu.async_copy(t2_h.at[isl], g_v, sem, add=True).wait()
            pltpu.async_copy(t3_h.at[isl], g_v, sem, add=True).wait()
            q = jnp.full((16,), 0.25, jnp.float32)
            for r in range(SB):
                for h in range(DH // 16):
                    sl = pl.ds(h * 16, 16)
                    g_v[r, sl] = g_v[r, sl] * q
            pltpu.sync_copy(
                g_v, out_h.at[pl.ds(c * B + s * _BPT + j * SB, SB)])

    w = s * NC + c
    for ids_h, emb_h, out_h in ((uid_h, uemb_h, uego_h),
                                (iid_h, iemb_h, pego_h),
                                (nid_h, iemb_h, nego_h)):
        pltpu.sync_copy(ids_h.at[pl.ds(w * _BPW, _BPW)], idx2_v)
        pltpu.async_copy(emb_h.at[idx2_v], ego_v, sem).wait()
        pltpu.sync_copy(ego_v, out_h.at[pl.ds(w * _BPW, _BPW)])


def _loss_body(ue_ref, pe_ref, ne_ref, uego_ref, pego_ref, nego_ref, out_ref):
    ue = ue_ref[...]
    pe = pe_ref[...]
    ne = ne_ref[...]
    pos = jnp.sum(ue * pe, axis=1)
    neg = jnp.sum(ue * ne, axis=1)
    x = neg - pos
    sp = jnp.maximum(x, 0.0) + jnp.log1p(jnp.exp(-jnp.abs(x)))
    bpr = jnp.mean(sp)
    reg = 0.5 * (jnp.sum(uego_ref[...] ** 2) + jnp.sum(pego_ref[...] ** 2)
                 + jnp.sum(nego_ref[...] ** 2)) / B
    out_ref[...] = jnp.reshape(bpr + LMBD * reg, (1, 1))


_loss_tc = pl.pallas_call(
    _loss_body,
    out_shape=jax.ShapeDtypeStruct((1, 1), jnp.float32),
)


@functools.lru_cache(maxsize=1)
def _build_sc_kernels():
    mesh = plsc.VectorSubcoreMesh(core_axis_name="c", subcore_axis_name="s",
                                  num_cores=NC, num_subcores=NS)
    params = pltpu.CompilerParams(use_tc_tiling_on_sc=False)
    propagate = pl.kernel(
        _propagate_layer_body,
        out_type=jax.ShapeDtypeStruct((2 * N_PAD, DH), jnp.float32),
        mesh=mesh,
        compiler_params=params,
        scratch_types=[
            pltpu.VMEM((CHUNK,), jnp.int32),
            pltpu.VMEM((SB_PER_CHUNK, SB), jnp.int32),
            pltpu.VMEM((CHUNK,), jnp.float32),
            (pltpu.VMEM((SB, DH), jnp.float32),
             pltpu.VMEM((SB, DH), jnp.float32),
             pltpu.VMEM((SB, DH), jnp.float32),
             pltpu.VMEM((SB, DH), jnp.float32)),
            (pltpu.VMEM((SB, DH), jnp.float32),
             pltpu.VMEM((SB, DH), jnp.float32)),
            (pltpu.SemaphoreType.DMA, pltpu.SemaphoreType.DMA,
             pltpu.SemaphoreType.DMA, pltpu.SemaphoreType.DMA,
             pltpu.SemaphoreType.DMA, pltpu.SemaphoreType.DMA),
            pltpu.VMEM_SHARED((N_PAD, DH), jnp.float32),
        ],
    )
    gather_stage = pl.kernel(
        _gather_stage_body,
        out_type=(
            jax.ShapeDtypeStruct((2 * B, DH), jnp.float32),
            jax.ShapeDtypeStruct((2 * B, DH), jnp.float32),
            jax.ShapeDtypeStruct((2 * B, DH), jnp.float32),
            jax.ShapeDtypeStruct((B, D), jnp.float32),
            jax.ShapeDtypeStruct((B, D), jnp.float32),
            jax.ShapeDtypeStruct((B, D), jnp.float32),
        ),
        mesh=mesh,
        compiler_params=params,
        scratch_types=[
            pltpu.VMEM((_BPT,), jnp.int32),
            pltpu.VMEM((SB, DH), jnp.float32),
            pltpu.VMEM((_BPW,), jnp.int32),
            pltpu.VMEM((_BPW, D), jnp.float32),
            pltpu.SemaphoreType.DMA,
        ],
    )
    return propagate, gather_stage


@jax.jit
def kernel(user_emb, item_emb, edge_val, edge_src, edge_dst,
           user_id, item_id, neg_item_id):
    all0 = jnp.concatenate(
        [user_emb, item_emb, jnp.zeros((N_PAD - N, D), jnp.float32)], axis=0)
    t0 = jnp.concatenate([all0[:, :DH], all0[:, DH:]], axis=0)

    es = edge_src.astype(jnp.int32)
    ed = edge_dst.astype(jnp.int32)
    key = (ed >= (N_PAD // 2)).astype(jnp.int32)
    nlo = E - jnp.sum(key)
    pos = jnp.where(key == 0, jnp.cumsum(1 - key) - 1,
                    nlo + jnp.cumsum(key) - 1)
    inv = jnp.zeros((E,), jnp.int32).at[pos].set(
        jnp.arange(E, dtype=jnp.int32), unique_indices=True,
        mode="promise_in_bounds")
    es = es[inv]
    ed = ed[inv]
    ev = edge_val[inv]

    pad = E_PAD - E
    src = jnp.concatenate([es, jnp.zeros((pad,), jnp.int32)])
    dst = jnp.concatenate([ed, jnp.zeros((pad,), jnp.int32)])
    val = jnp.concatenate([ev, jnp.zeros((pad,), jnp.float32)])
    src1 = src + N_PAD
    dst2d = dst.reshape(E_PAD // SB, SB)
    zeros = jnp.zeros((N_PAD, DH), jnp.float32)

    propagate, gather_stage = _build_sc_kernels()
    t1 = propagate(t0, src, src1, dst2d, val, zeros)
    t2 = propagate(t1, src, src1, dst2d, val, zeros)
    t3 = propagate(t2, src, src1, dst2d, val, zeros)

    uid = user_id.astype(jnp.int32)
    iid = item_id.astype(jnp.int32)
    nid = neg_item_id.astype(jnp.int32)
    uidx = jnp.stack([uid, uid + N_PAD])
    iidx = jnp.stack([iid + USER_NUM, iid + USER_NUM + N_PAD])
    nidx = jnp.stack([nid + USER_NUM, nid + USER_NUM + N_PAD])

    ue2, pe2, ne2, uego, pego, nego = gather_stage(
        t0, t1, t2, t3, uidx, iidx, nidx, uid, iid, nid, user_emb, item_emb)

    def _assemble(x2):
        return x2.reshape(2, B, DH).transpose(1, 0, 2).reshape(B, D)

    ue = _assemble(ue2)
    pe = _assemble(pe2)
    ne = _assemble(ne2)

    loss = _loss_tc(ue, pe, ne, uego, pego, nego)
    return loss[0, 0]


# speedup vs baseline: 3.1145x; 3.1145x over previous
import functools
import jax
import jax.numpy as jnp
from jax import lax
from jax.experimental import pallas as pl
from jax.experimental.pallas import tpu as pltpu
from jax.experimental.pallas import tpu_sc as plsc

USER_NUM = 20000
ITEM_NUM = 30000
N = USER_NUM + ITEM_NUM
E = 800000
D = 64
DH = 32
B = 4096
N_LAYERS = 3
LMBD = 1e-4

NC = 2
NS = 16
SB = 128
SB_PER_CHUNK = 16
CHUNK = SB * SB_PER_CHUNK
N_CHUNKS = 25
EC = CHUNK * N_CHUNKS
E_PAD = EC * NS

N_PAD = 50048
ROWS_PER_TILE = N_PAD // NS


def _lane_broadcast(v, e):
    idx = jnp.full((16, 1), e, dtype=jnp.int32)
    dnums = lax.GatherDimensionNumbers(
        offset_dims=(), collapsed_slice_dims=(0,), start_index_map=(0,))
    return lax.gather(v, idx, dnums, (1,),
                      mode=lax.GatherScatterMode.PROMISE_IN_BOUNDS)


def _scale_rows(rows_ref, msg_ref, val_ref, vbase):
    for g in range(SB // 16):
        v = val_ref[pl.ds(vbase + g * 16, 16)]
        for e in range(16):
            b = _lane_broadcast(v, e)
            r = g * 16 + e
            for h in range(DH // 16):
                sl = pl.ds(h * 16, 16)
                msg_ref[r, sl] = rows_ref[r, sl] * b


def _propagate_layer_body(table_h, src0_h, src1_h, dst2d_h, val_h, zeros_h,
                          out_h, src_v, dst_v, val_v, rows_v, msg_v, sem,
                          acc_sh):
    c = lax.axis_index("c")
    s = lax.axis_index("s")
    row0 = s * ROWS_PER_TILE

    pltpu.sync_copy(zeros_h.at[pl.ds(row0, ROWS_PER_TILE)],
                    acc_sh.at[pl.ds(row0, ROWS_PER_TILE)])
    plsc.subcore_barrier()

    ebase = s * EC
    msg_a, msg_b = msg_v
    gsems = sem[:4]
    ssem_a, ssem_b = sem[4], sem[5]
    msgs = (msg_a, msg_b)
    ssems = (ssem_a, ssem_b)

    def _gather(j, rowsb, gsem):
        pltpu.async_copy(table_h.at[src_v.at[pl.ds(j * SB, SB)]], rowsb, gsem)

    def _wait_gather(rowsb, gsem):
        pltpu.make_async_copy(table_h.at[src_v.at[pl.ds(0, SB)]], rowsb,
                              gsem).wait()

    def _scatter(j, msgb, ssem):
        pltpu.async_copy(msgb, acc_sh.at[dst_v.at[j]], ssem, add=True)

    def _wait_scatter(msgb, ssem):
        pltpu.make_async_copy(msgb, acc_sh.at[dst_v.at[0]], ssem).wait()

    @pl.loop(0, N_CHUNKS)
    def _chunk(i):
        @pl.when(i > 0)
        def _():
            _wait_scatter(msg_a, ssem_a)
            _wait_scatter(msg_b, ssem_b)

        base = pl.multiple_of(ebase + i * CHUNK, CHUNK)
        row_base = pl.multiple_of(base // SB, SB_PER_CHUNK)

        @pl.when(c == 0)
        def _():
            pltpu.sync_copy(src0_h.at[pl.ds(base, CHUNK)], src_v)

        @pl.when(c == 1)
        def _():
            pltpu.sync_copy(src1_h.at[pl.ds(base, CHUNK)], src_v)

        pltpu.sync_copy(dst2d_h.at[pl.ds(row_base, SB_PER_CHUNK)], dst_v)
        pltpu.sync_copy(val_h.at[pl.ds(base, CHUNK)], val_v)

        for u in range(3):
            _gather(u, rows_v[u], gsems[u])

        @pl.loop(0, SB_PER_CHUNK // 4)
        def _quad(k):
            j = k * 4
            for u in range(4):
                jj = j + u
                _wait_gather(rows_v[u], gsems[u])
                if u < 2:
                    @pl.when(k > 0)
                    def _():
                        _wait_scatter(msgs[u % 2], ssems[u % 2])
                else:
                    _wait_scatter(msgs[u % 2], ssems[u % 2])
                _scale_rows(rows_v[u], msgs[u % 2], val_v, jj * SB)
                _scatter(jj, msgs[u % 2], ssems[u % 2])
                if u == 0:
                    _gather(jj + 3, rows_v[3], gsems[3])
                else:
                    @pl.when(k < SB_PER_CHUNK // 4 - 1)
                    def _():
                        _gather(jj + 3, rows_v[(u + 3) % 4],
                                gsems[(u + 3) % 4])

    _wait_scatter(msg_a, ssem_a)
    _wait_scatter(msg_b, ssem_b)
    plsc.subcore_barrier()

    pltpu.sync_copy(acc_sh.at[pl.ds(row0, ROWS_PER_TILE)],
                    out_h.at[pl.ds(c * N_PAD + row0, ROWS_PER_TILE)])


_BPT = B // NS
_BPW = B // (NC * NS)


def _gather_stage_body(t0_h, t1_h, t2_h, t3_h, uidx_h, iidx_h, nidx_h,
                       uid_h, iid_h, nid_h, uemb_h, iemb_h,
                       ue_h, pe_h, ne_h, uego_h, pego_h, nego_h,
                       idx_v, g_v, idx2_v, ego_v, sem):
    c = lax.axis_index("c")
    s = lax.axis_index("s")

    for set_idx, ids_h, out_h in ((0, uidx_h, ue_h), (1, iidx_h, pe_h),
                                  (2, nidx_h, ne_h)):
        pltpu.sync_copy(ids_h.at[c, pl.ds(s * _BPT, _BPT)], idx_v)

        @pl.loop(0, _BPT // SB)
        def _blk(j):
            isl = idx_v.at[pl.ds(j * SB, SB)]
            pltpu.async_copy(t0_h.at[isl], g_v, sem).wait()
            pltpu.async_copy(t1_h.at[isl], g_v, sem, add=True).wait()
            pltpu.async_copy(t2_h.at[isl], g_v, sem, add=True).wait()
            pltpu.async_copy(t3_h.at[isl], g_v, sem, add=True).wait()
            q = jnp.full((16,), 0.25, jnp.float32)
            for r in range(SB):
                for h in range(DH // 16):
                    sl = pl.ds(h * 16, 16)
                    g_v[r, sl] = g_v[r, sl] * q
            pltpu.sync_copy(
                g_v, out_h.at[pl.ds(c * B + s * _BPT + j * SB, SB)])

    w = s * NC + c
    for ids_h, emb_h, out_h in ((uid_h, uemb_h, uego_h),
                                (iid_h, iemb_h, pego_h),
                                (nid_h, iemb_h, nego_h)):
        pltpu.sync_copy(ids_h.at[pl.ds(w * _BPW, _BPW)], idx2_v)
        pltpu.async_copy(emb_h.at[idx2_v], ego_v, sem).wait()
        pltpu.sync_copy(ego_v, out_h.at[pl.ds(w * _BPW, _BPW)])


def _loss_body(ue_ref, pe_ref, ne_ref, uego_ref, pego_ref, nego_ref, out_ref):
    ue = ue_ref[...]
    pe = pe_ref[...]
    ne = ne_ref[...]
    pos = jnp.sum(ue * pe, axis=1)
    neg = jnp.sum(ue * ne, axis=1)
    x = neg - pos
    sp = jnp.maximum(x, 0.0) + jnp.log1p(jnp.exp(-jnp.abs(x)))
    bpr = jnp.mean(sp)
    reg = 0.5 * (jnp.sum(uego_ref[...] ** 2) + jnp.sum(pego_ref[...] ** 2)
                 + jnp.sum(nego_ref[...] ** 2)) / B
    out_ref[...] = jnp.reshape(bpr + LMBD * reg, (1, 1))


_loss_tc = pl.pallas_call(
    _loss_body,
    out_shape=jax.ShapeDtypeStruct((1, 1), jnp.float32),
)


@functools.lru_cache(maxsize=1)
def _build_sc_kernels():
    mesh = plsc.VectorSubcoreMesh(core_axis_name="c", subcore_axis_name="s",
                                  num_cores=NC, num_subcores=NS)
    params = pltpu.CompilerParams(use_tc_tiling_on_sc=False)
    propagate = pl.kernel(
        _propagate_layer_body,
        out_type=jax.ShapeDtypeStruct((2 * N_PAD, DH), jnp.float32),
        mesh=mesh,
        compiler_params=params,
        scratch_types=[
            pltpu.VMEM((CHUNK,), jnp.int32),
            pltpu.VMEM((SB_PER_CHUNK, SB), jnp.int32),
            pltpu.VMEM((CHUNK,), jnp.float32),
            (pltpu.VMEM((SB, DH), jnp.float32),
             pltpu.VMEM((SB, DH), jnp.float32),
             pltpu.VMEM((SB, DH), jnp.float32),
             pltpu.VMEM((SB, DH), jnp.float32)),
            (pltpu.VMEM((SB, DH), jnp.float32),
             pltpu.VMEM((SB, DH), jnp.float32)),
            (pltpu.SemaphoreType.DMA, pltpu.SemaphoreType.DMA,
             pltpu.SemaphoreType.DMA, pltpu.SemaphoreType.DMA,
             pltpu.SemaphoreType.DMA, pltpu.SemaphoreType.DMA),
            pltpu.VMEM_SHARED((N_PAD, DH), jnp.float32),
        ],
    )
    gather_stage = pl.kernel(
        _gather_stage_body,
        out_type=(
            jax.ShapeDtypeStruct((2 * B, DH), jnp.float32),
            jax.ShapeDtypeStruct((2 * B, DH), jnp.float32),
            jax.ShapeDtypeStruct((2 * B, DH), jnp.float32),
            jax.ShapeDtypeStruct((B, D), jnp.float32),
            jax.ShapeDtypeStruct((B, D), jnp.float32),
            jax.ShapeDtypeStruct((B, D), jnp.float32),
        ),
        mesh=mesh,
        compiler_params=params,
        scratch_types=[
            pltpu.VMEM((_BPT,), jnp.int32),
            pltpu.VMEM((SB, DH), jnp.float32),
            pltpu.VMEM((_BPW,), jnp.int32),
            pltpu.VMEM((_BPW, D), jnp.float32),
            pltpu.SemaphoreType.DMA,
        ],
    )
    return propagate, gather_stage


@jax.jit
def kernel(user_emb, item_emb, edge_val, edge_src, edge_dst,
           user_id, item_id, neg_item_id):
    all0 = jnp.concatenate(
        [user_emb, item_emb, jnp.zeros((N_PAD - N, D), jnp.float32)], axis=0)
    t0 = jnp.concatenate([all0[:, :DH], all0[:, DH:]], axis=0)

    pad = E_PAD - E
    src = jnp.concatenate([edge_src.astype(jnp.int32),
                           jnp.zeros((pad,), jnp.int32)])
    dst = jnp.concatenate([edge_dst.astype(jnp.int32),
                           jnp.zeros((pad,), jnp.int32)])
    val = jnp.concatenate([edge_val, jnp.zeros((pad,), jnp.float32)])
    src1 = src + N_PAD
    dst2d = dst.reshape(E_PAD // SB, SB)
    zeros = jnp.zeros((N_PAD, DH), jnp.float32)

    propagate, gather_stage = _build_sc_kernels()
    t1 = propagate(t0, src, src1, dst2d, val, zeros)
    t2 = propagate(t1, src, src1, dst2d, val, zeros)
    t3 = propagate(t2, src, src1, dst2d, val, zeros)

    uid = user_id.astype(jnp.int32)
    iid = item_id.astype(jnp.int32)
    nid = neg_item_id.astype(jnp.int32)
    uidx = jnp.stack([uid, uid + N_PAD])
    iidx = jnp.stack([iid + USER_NUM, iid + USER_NUM + N_PAD])
    nidx = jnp.stack([nid + USER_NUM, nid + USER_NUM + N_PAD])

    ue2, pe2, ne2, uego, pego, nego = gather_stage(
        t0, t1, t2, t3, uidx, iidx, nidx, uid, iid, nid, user_emb, item_emb)

    def _assemble(x2):
        return x2.reshape(2, B, DH).transpose(1, 0, 2).reshape(B, D)

    ue = _assemble(ue2)
    pe = _assemble(pe2)
    ne = _assemble(ne2)

    loss = _loss_tc(ue, pe, ne, uego, pego, nego)
    return loss[0, 0]


# speedup vs baseline: 3.2055x; 1.0292x over previous
import functools
import jax
import jax.numpy as jnp
from jax import lax
from jax.experimental import pallas as pl
from jax.experimental.pallas import tpu as pltpu
from jax.experimental.pallas import tpu_sc as plsc

USER_NUM = 20000
ITEM_NUM = 30000
N = USER_NUM + ITEM_NUM
E = 800000
D = 64
DH = 32
B = 4096
N_LAYERS = 3
LMBD = 1e-4

NC = 2
NS = 16
SB = 128
SB_PER_CHUNK = 16
CHUNK = SB * SB_PER_CHUNK
N_CHUNKS = 25
EC = CHUNK * N_CHUNKS
E_PAD = EC * NS

N_PAD = 50048
ROWS_PER_TILE = N_PAD // NS


def _lane_broadcast(v, e):
    idx = jnp.full((16, 1), e, dtype=jnp.int32)
    dnums = lax.GatherDimensionNumbers(
        offset_dims=(), collapsed_slice_dims=(0,), start_index_map=(0,))
    return lax.gather(v, idx, dnums, (1,),
                      mode=lax.GatherScatterMode.PROMISE_IN_BOUNDS)


def _scale_rows(rows_ref, msg_ref, val_ref, vbase):
    for g in range(SB // 16):
        v = val_ref[pl.ds(vbase + g * 16, 16)]
        for e in range(16):
            b = _lane_broadcast(v, e)
            r = g * 16 + e
            for h in range(DH // 16):
                sl = pl.ds(h * 16, 16)
                msg_ref[r, sl] = rows_ref[r, sl] * b


def _propagate3_body(t0_h, src0_h, src1_h, dst2d_h, val_h, zeros_h,
                     t1_h, t2_h, t3_h, src_v, dst_v, val_v, rows_v, msg_v,
                     sem, acc_sh):
    c = lax.axis_index("c")
    s = lax.axis_index("s")
    row0 = s * ROWS_PER_TILE
    ebase = s * EC
    msg_a, msg_b = msg_v
    gsems = sem[:4]
    ssem_a, ssem_b = sem[4], sem[5]
    msgs = (msg_a, msg_b)
    ssems = (ssem_a, ssem_b)

    def _scatter(j, msgb, ssem):
        pltpu.async_copy(msgb, acc_sh.at[dst_v.at[j]], ssem, add=True)

    def _wait_scatter(msgb, ssem):
        pltpu.make_async_copy(msgb, acc_sh.at[dst_v.at[0]], ssem).wait()

    def _layer(table_h, out_h):
        def _gather(j, rowsb, gsem):
            pltpu.async_copy(table_h.at[src_v.at[pl.ds(j * SB, SB)]], rowsb,
                             gsem)

        def _wait_gather(rowsb, gsem):
            pltpu.make_async_copy(table_h.at[src_v.at[pl.ds(0, SB)]], rowsb,
                                  gsem).wait()

        pltpu.sync_copy(zeros_h.at[pl.ds(row0, ROWS_PER_TILE)],
                        acc_sh.at[pl.ds(row0, ROWS_PER_TILE)])
        plsc.subcore_barrier()

        @pl.loop(0, N_CHUNKS)
        def _chunk(i):
            @pl.when(i > 0)
            def _():
                _wait_scatter(msg_a, ssem_a)
                _wait_scatter(msg_b, ssem_b)

            base = pl.multiple_of(ebase + i * CHUNK, CHUNK)
            row_base = pl.multiple_of(base // SB, SB_PER_CHUNK)

            @pl.when(c == 0)
            def _():
                pltpu.sync_copy(src0_h.at[pl.ds(base, CHUNK)], src_v)

            @pl.when(c == 1)
            def _():
                pltpu.sync_copy(src1_h.at[pl.ds(base, CHUNK)], src_v)

            pltpu.sync_copy(dst2d_h.at[pl.ds(row_base, SB_PER_CHUNK)], dst_v)
            pltpu.sync_copy(val_h.at[pl.ds(base, CHUNK)], val_v)

            for u in range(3):
                _gather(u, rows_v[u], gsems[u])

            @pl.loop(0, SB_PER_CHUNK // 4)
            def _quad(k):
                j = k * 4
                for u in range(4):
                    jj = j + u
                    _wait_gather(rows_v[u], gsems[u])
                    if u < 2:
                        @pl.when(k > 0)
                        def _():
                            _wait_scatter(msgs[u % 2], ssems[u % 2])
                    else:
                        _wait_scatter(msgs[u % 2], ssems[u % 2])
                    _scale_rows(rows_v[u], msgs[u % 2], val_v, jj * SB)
                    _scatter(jj, msgs[u % 2], ssems[u % 2])
                    if u == 0:
                        _gather(jj + 3, rows_v[3], gsems[3])
                    else:
                        @pl.when(k < SB_PER_CHUNK // 4 - 1)
                        def _():
                            _gather(jj + 3, rows_v[(u + 3) % 4],
                                    gsems[(u + 3) % 4])

        _wait_scatter(msg_a, ssem_a)
        _wait_scatter(msg_b, ssem_b)
        plsc.subcore_barrier()

        pltpu.sync_copy(acc_sh.at[pl.ds(row0, ROWS_PER_TILE)],
                        out_h.at[pl.ds(c * N_PAD + row0, ROWS_PER_TILE)])

    _layer(t0_h, t1_h)
    _layer(t1_h, t2_h)
    _layer(t2_h, t3_h)


_BPT = B // NS
_BPW = B // (NC * NS)


def _gather_stage_body(t0_h, t1_h, t2_h, t3_h, uidx_h, iidx_h, nidx_h,
                       uid_h, iid_h, nid_h, uemb_h, iemb_h,
                       ue_h, pe_h, ne_h, uego_h, pego_h, nego_h,
                       idx_v, g_v, idx2_v, ego_v, sem):
    c = lax.axis_index("c")
    s = lax.axis_index("s")

    for set_idx, ids_h, out_h in ((0, uidx_h, ue_h), (1, iidx_h, pe_h),
                                  (2, nidx_h, ne_h)):
        pltpu.sync_copy(ids_h.at[c, pl.ds(s * _BPT, _BPT)], idx_v)

        @pl.loop(0, _BPT // SB)
        def _blk(j):
            isl = idx_v.at[pl.ds(j * SB, SB)]
            pltpu.async_copy(t0_h.at[isl], g_v, sem).wait()
            pltpu.async_copy(t1_h.at[isl], g_v, sem, add=True).wait()
            pltpu.async_copy(t2_h.at[isl], g_v, sem, add=True).wait()
            pltpu.async_copy(t3_h.at[isl], g_v, sem, add=True).wait()
            q = jnp.full((16,), 0.25, jnp.float32)
            for r in range(SB):
                for h in range(DH // 16):
                    sl = pl.ds(h * 16, 16)
                    g_v[r, sl] = g_v[r, sl] * q
            pltpu.sync_copy(
                g_v, out_h.at[pl.ds(c * B + s * _BPT + j * SB, SB)])

    w = s * NC + c
    for ids_h, emb_h, out_h in ((uid_h, uemb_h, uego_h),
                                (iid_h, iemb_h, pego_h),
                                (nid_h, iemb_h, nego_h)):
        pltpu.sync_copy(ids_h.at[pl.ds(w * _BPW, _BPW)], idx2_v)
        pltpu.async_copy(emb_h.at[idx2_v], ego_v, sem).wait()
        pltpu.sync_copy(ego_v, out_h.at[pl.ds(w * _BPW, _BPW)])


def _loss_body(ue_ref, pe_ref, ne_ref, uego_ref, pego_ref, nego_ref, out_ref):
    ue = ue_ref[...]
    pe = pe_ref[...]
    ne = ne_ref[...]
    pos = jnp.sum(ue * pe, axis=1)
    neg = jnp.sum(ue * ne, axis=1)
    x = neg - pos
    sp = jnp.maximum(x, 0.0) + jnp.log1p(jnp.exp(-jnp.abs(x)))
    bpr = jnp.mean(sp)
    reg = 0.5 * (jnp.sum(uego_ref[...] ** 2) + jnp.sum(pego_ref[...] ** 2)
                 + jnp.sum(nego_ref[...] ** 2)) / B
    out_ref[...] = jnp.reshape(bpr + LMBD * reg, (1, 1))


_loss_tc = pl.pallas_call(
    _loss_body,
    out_shape=jax.ShapeDtypeStruct((1, 1), jnp.float32),
)


@functools.lru_cache(maxsize=1)
def _build_sc_kernels():
    mesh = plsc.VectorSubcoreMesh(core_axis_name="c", subcore_axis_name="s",
                                  num_cores=NC, num_subcores=NS)
    params = pltpu.CompilerParams(use_tc_tiling_on_sc=False)
    propagate = pl.kernel(
        _propagate3_body,
        out_type=(jax.ShapeDtypeStruct((2 * N_PAD, DH), jnp.float32),
                  jax.ShapeDtypeStruct((2 * N_PAD, DH), jnp.float32),
                  jax.ShapeDtypeStruct((2 * N_PAD, DH), jnp.float32)),
        mesh=mesh,
        compiler_params=params,
        scratch_types=[
            pltpu.VMEM((CHUNK,), jnp.int32),
            pltpu.VMEM((SB_PER_CHUNK, SB), jnp.int32),
            pltpu.VMEM((CHUNK,), jnp.float32),
            (pltpu.VMEM((SB, DH), jnp.float32),
             pltpu.VMEM((SB, DH), jnp.float32),
             pltpu.VMEM((SB, DH), jnp.float32),
             pltpu.VMEM((SB, DH), jnp.float32)),
            (pltpu.VMEM((SB, DH), jnp.float32),
             pltpu.VMEM((SB, DH), jnp.float32)),
            (pltpu.SemaphoreType.DMA, pltpu.SemaphoreType.DMA,
             pltpu.SemaphoreType.DMA, pltpu.SemaphoreType.DMA,
             pltpu.SemaphoreType.DMA, pltpu.SemaphoreType.DMA),
            pltpu.VMEM_SHARED((N_PAD, DH), jnp.float32),
        ],
    )
    gather_stage = pl.kernel(
        _gather_stage_body,
        out_type=(
            jax.ShapeDtypeStruct((2 * B, DH), jnp.float32),
            jax.ShapeDtypeStruct((2 * B, DH), jnp.float32),
            jax.ShapeDtypeStruct((2 * B, DH), jnp.float32),
            jax.ShapeDtypeStruct((B, D), jnp.float32),
            jax.ShapeDtypeStruct((B, D), jnp.float32),
            jax.ShapeDtypeStruct((B, D), jnp.float32),
        ),
        mesh=mesh,
        compiler_params=params,
        scratch_types=[
            pltpu.VMEM((_BPT,), jnp.int32),
            pltpu.VMEM((SB, DH), jnp.float32),
            pltpu.VMEM((_BPW,), jnp.int32),
            pltpu.VMEM((_BPW, D), jnp.float32),
            pltpu.SemaphoreType.DMA,
        ],
    )
    return propagate, gather_stage


@jax.jit
def kernel(user_emb, item_emb, edge_val, edge_src, edge_dst,
           user_id, item_id, neg_item_id):
    all0 = jnp.concatenate(
        [user_emb, item_emb, jnp.zeros((N_PAD - N, D), jnp.float32)], axis=0)
    t0 = jnp.concatenate([all0[:, :DH], all0[:, DH:]], axis=0)

    pad = E_PAD - E
    src = jnp.concatenate([edge_src.astype(jnp.int32),
                           jnp.zeros((pad,), jnp.int32)])
    dst = jnp.concatenate([edge_dst.astype(jnp.int32),
                           jnp.zeros((pad,), jnp.int32)])
    val = jnp.concatenate([edge_val, jnp.zeros((pad,), jnp.float32)])
    src1 = src + N_PAD
    dst2d = dst.reshape(E_PAD // SB, SB)
    zeros = jnp.zeros((N_PAD, DH), jnp.float32)

    propagate, gather_stage = _build_sc_kernels()
    t1, t2, t3 = propagate(t0, src, src1, dst2d, val, zeros)

    uid = user_id.astype(jnp.int32)
    iid = item_id.astype(jnp.int32)
    nid = neg_item_id.astype(jnp.int32)
    uidx = jnp.stack([uid, uid + N_PAD])
    iidx = jnp.stack([iid + USER_NUM, iid + USER_NUM + N_PAD])
    nidx = jnp.stack([nid + USER_NUM, nid + USER_NUM + N_PAD])

    ue2, pe2, ne2, uego, pego, nego = gather_stage(
        t0, t1, t2, t3, uidx, iidx, nidx, uid, iid, nid, user_emb, item_emb)

    def _assemble(x2):
        return x2.reshape(2, B, DH).transpose(1, 0, 2).reshape(B, D)

    ue = _assemble(ue2)
    pe = _assemble(pe2)
    ne = _assemble(ne2)

    loss = _loss_tc(ue, pe, ne, uego, pego, nego)
    return loss[0, 0]
